# Initial kernel scaffold; baseline (speedup 1.0000x reference)
import jax
import jax.numpy as jnp
from jax.experimental import pallas as pl


def kernel(x_0, x_1, x_2, hodge_laplacian_0, hodge_laplacian_1, hodge_laplacian_2, incidence_1, incidence_2, batch, batch_1, y, W0, W1, W2, Wa1, Wa2, gn1_gamma, gn1_beta, gn1_alpha, gn2_gamma, gn2_beta, gn2_alpha):
    raise NotImplementedError("write your pallas kernel here")


# speedup vs baseline: 2.9490x; 2.9490x over previous
import functools

import jax
import jax.numpy as jnp
from jax.experimental import pallas as pl
from jax.experimental.pallas import tpu as pltpu

_CDT = jnp.bfloat16


def _row_inv_body(m_ref, inv_ref):
    s = jnp.sum(jnp.abs(m_ref[...]), axis=1)
    safe = jnp.where(s != 0, s, 1.0)
    inv_ref[...] = jnp.where(s != 0, 1.0 / jnp.sqrt(safe), 0.0)


def _row_inv(m, bm=512):
    n, k = m.shape
    return pl.pallas_call(
        _row_inv_body,
        grid=(n // bm,),
        in_specs=[pl.BlockSpec((bm, k), lambda i: (i, 0))],
        out_specs=pl.BlockSpec((bm,), lambda i: (i,)),
        out_shape=jax.ShapeDtypeStruct((n,), jnp.float32),
    )(m)


def _conv_body(m_ref, x_ref, w_ref, invk_ref, invm_ref, out_ref, z_scr):
    @pl.when(pl.program_id(0) == 0)
    def _():
        z = jnp.dot(x_ref[...].astype(_CDT), w_ref[...].astype(_CDT),
                    preferred_element_type=jnp.float32)
        z_scr[...] = (invk_ref[...][:, None] * z).astype(_CDT)

    acc = jnp.dot(m_ref[...].astype(_CDT), z_scr[...],
                  preferred_element_type=jnp.float32)
    out_ref[...] = jnp.maximum(invm_ref[...][:, None] * acc, 0.0)


def _conv(m, x, w, inv, bm=512):
    n, k = m.shape
    c = x.shape[1]
    return pl.pallas_call(
        _conv_body,
        grid=(n // bm,),
        in_specs=[
            pl.BlockSpec((bm, k), lambda i: (i, 0)),
            pl.BlockSpec((k, c), lambda i: (0, 0)),
            pl.BlockSpec((c, c), lambda i: (0, 0)),
            pl.BlockSpec((k,), lambda i: (0,)),
            pl.BlockSpec((bm,), lambda i: (i,)),
        ],
        out_specs=pl.BlockSpec((bm, c), lambda i: (i, 0)),
        out_shape=jax.ShapeDtypeStruct((n, c), jnp.float32),
        scratch_shapes=[pltpu.VMEM((k, c), _CDT)],
    )(m, x, w, inv, inv)


def _agg_body(m_ref, x_ref, w_ref, add_ref, out_ref, z_scr):
    @pl.when(pl.program_id(0) == 0)
    def _():
        z_scr[...] = jnp.dot(x_ref[...].astype(_CDT), w_ref[...].astype(_CDT),
                             preferred_element_type=jnp.float32).astype(_CDT)

    acc = jnp.dot(m_ref[...].astype(_CDT), z_scr[...],
                  preferred_element_type=jnp.float32)
    out_ref[...] = add_ref[...] + acc


def _agg(m, x, w, add, bm=512):
    n, k = m.shape
    c = x.shape[1]
    return pl.pallas_call(
        _agg_body,
        grid=(n // bm,),
        in_specs=[
            pl.BlockSpec((bm, k), lambda i: (i, 0)),
            pl.BlockSpec((k, c), lambda i: (0, 0)),
            pl.BlockSpec((c, c), lambda i: (0, 0)),
            pl.BlockSpec((bm, c), lambda i: (i, 0)),
        ],
        out_specs=pl.BlockSpec((bm, c), lambda i: (i, 0)),
        out_shape=jax.ShapeDtypeStruct((n, c), jnp.float32),
        scratch_shapes=[pltpu.VMEM((k, c), _CDT)],
    )(m, x, w, add)


def _graph_norm_body(g, eps, x_ref, b_ref, gamma_ref, beta_ref, alpha_ref,
                     out_ref):
    x = x_ref[...]
    b = b_ref[...]
    n = x.shape[0]
    seg_rows = jax.lax.broadcasted_iota(jnp.int32, (g, n), 0)
    sg = (seg_rows == b[None, :]).astype(jnp.float32)
    st = (b[:, None] == jax.lax.broadcasted_iota(jnp.int32, (n, g), 1)
          ).astype(jnp.float32)
    counts = jnp.maximum(jnp.sum(sg, axis=1), 1.0)[:, None]
    mean = jnp.dot(sg, x, preferred_element_type=jnp.float32) / counts
    xc = x - alpha_ref[...][None, :] * jnp.dot(
        st, mean, preferred_element_type=jnp.float32)
    var = jnp.dot(sg, xc * xc, preferred_element_type=jnp.float32) / counts
    rstd = 1.0 / jnp.sqrt(var + eps)
    out_ref[...] = (xc * jnp.dot(st, rstd, preferred_element_type=jnp.float32)
                    * gamma_ref[...][None, :] + beta_ref[...][None, :])


def _graph_norm(x, batch, gamma, beta, alpha, g, eps=1e-5):
    n, c = x.shape
    return pl.pallas_call(
        functools.partial(_graph_norm_body, g, eps),
        out_shape=jax.ShapeDtypeStruct((n, c), jnp.float32),
    )(x, batch.astype(jnp.int32), gamma, beta, alpha)


def kernel(x_0, x_1, x_2, hodge_laplacian_0, hodge_laplacian_1,
           hodge_laplacian_2, incidence_1, incidence_2, batch, batch_1, y,
           W0, W1, W2, Wa1, Wa2,
           gn1_gamma, gn1_beta, gn1_alpha, gn2_gamma, gn2_beta, gn2_alpha):
    g = y.shape[0]
    inv0 = _row_inv(hodge_laplacian_0)
    inv1 = _row_inv(hodge_laplacian_1)
    inv2 = _row_inv(hodge_laplacian_2)
    x0b = _conv(hodge_laplacian_0, x_0, W0, inv0)
    x1b = _conv(hodge_laplacian_1, x_1, W1, inv1)
    x2b = _conv(hodge_laplacian_2, x_2, W2, inv2)
    x_2_out = x2b
    x1in = _agg(incidence_2, x_2_out, Wa1, x1b)
    x_1_out = _graph_norm(x1in, batch_1, gn1_gamma, gn1_beta, gn1_alpha, g)
    x0in = _agg(incidence_1, x_1_out, Wa2, x0b)
    x_0_out = _graph_norm(x0in, batch, gn2_gamma, gn2_beta, gn2_alpha, g)
    return (x_0_out, x_1_out, x_2_out)


# speedup vs baseline: 3.0833x; 1.0455x over previous
import functools

import jax
import jax.numpy as jnp
from jax.experimental import pallas as pl
from jax.experimental.pallas import tpu as pltpu

_CDT = jnp.bfloat16


def _conv_body(bm, m_ref, x_ref, w_ref, out_ref, s_scr, inv_scr, z_scr):
    p = pl.program_id(0)
    i = pl.program_id(1)

    @pl.when(p == 0)
    def _():
        s_scr[pl.ds(i * bm, bm)] = jnp.sum(jnp.abs(m_ref[...]), axis=1)

    @pl.when(p == 1)
    def _():
        @pl.when(i == 0)
        def _():
            s = s_scr[...]
            safe = jnp.where(s != 0, s, 1.0)
            inv = jnp.where(s != 0, 1.0 / jnp.sqrt(safe), 0.0)
            inv_scr[...] = inv
            z = jnp.dot(x_ref[...].astype(_CDT), w_ref[...].astype(_CDT),
                        preferred_element_type=jnp.float32)
            z_scr[...] = (inv[:, None] * z).astype(_CDT)

        acc = jnp.dot(m_ref[...].astype(_CDT), z_scr[...],
                      preferred_element_type=jnp.float32)
        invm = inv_scr[pl.ds(i * bm, bm)]
        out_ref[...] = jnp.maximum(invm[:, None] * acc, 0.0)


def _conv(m, x, w, bm=512):
    n = m.shape[0]
    c = x.shape[1]
    return pl.pallas_call(
        functools.partial(_conv_body, bm),
        grid=(2, n // bm),
        in_specs=[
            pl.BlockSpec((bm, n), lambda p, i: (i, 0)),
            pl.BlockSpec((n, c), lambda p, i: (0, 0)),
            pl.BlockSpec((c, c), lambda p, i: (0, 0)),
        ],
        out_specs=pl.BlockSpec((bm, c),
                               lambda p, i: (jnp.where(p == 0, 0, i), 0)),
        out_shape=jax.ShapeDtypeStruct((n, c), jnp.float32),
        scratch_shapes=[
            pltpu.VMEM((n,), jnp.float32),
            pltpu.VMEM((n,), jnp.float32),
            pltpu.VMEM((n, c), _CDT),
        ],
    )(m, x, w)


def _agg_stats_body(g, m_ref, x_ref, w_ref, add_ref, b_ref,
                    out_ref, cnt_ref, sum_ref, sq_ref, z_scr):
    i = pl.program_id(0)

    @pl.when(i == 0)
    def _():
        z_scr[...] = jnp.dot(x_ref[...].astype(_CDT), w_ref[...].astype(_CDT),
                             preferred_element_type=jnp.float32).astype(_CDT)
        cnt_ref[...] = jnp.zeros_like(cnt_ref)
        sum_ref[...] = jnp.zeros_like(sum_ref)
        sq_ref[...] = jnp.zeros_like(sq_ref)

    acc = jnp.dot(m_ref[...].astype(_CDT), z_scr[...],
                  preferred_element_type=jnp.float32)
    v = add_ref[...] + acc
    out_ref[...] = v
    b = b_ref[...]
    bm = b.shape[0]
    sg = (jax.lax.broadcasted_iota(jnp.int32, (g, bm), 0)
          == b[None, :]).astype(jnp.float32)
    cnt_ref[...] += jnp.sum(sg, axis=1)
    sum_ref[...] += jnp.dot(sg, v, preferred_element_type=jnp.float32)
    sq_ref[...] += jnp.dot(sg, v * v, preferred_element_type=jnp.float32)


def _agg_stats(m, x, w, add, batch, g, bm=512):
    n, k = m.shape
    c = x.shape[1]
    return pl.pallas_call(
        functools.partial(_agg_stats_body, g),
        grid=(n // bm,),
        in_specs=[
            pl.BlockSpec((bm, k), lambda i: (i, 0)),
            pl.BlockSpec((k, c), lambda i: (0, 0)),
            pl.BlockSpec((c, c), lambda i: (0, 0)),
            pl.BlockSpec((bm, c), lambda i: (i, 0)),
            pl.BlockSpec((bm,), lambda i: (i,)),
        ],
        out_specs=[
            pl.BlockSpec((bm, c), lambda i: (i, 0)),
            pl.BlockSpec((g,), lambda i: (0,)),
            pl.BlockSpec((g, c), lambda i: (0, 0)),
            pl.BlockSpec((g, c), lambda i: (0, 0)),
        ],
        out_shape=[
            jax.ShapeDtypeStruct((n, c), jnp.float32),
            jax.ShapeDtypeStruct((g,), jnp.float32),
            jax.ShapeDtypeStruct((g, c), jnp.float32),
            jax.ShapeDtypeStruct((g, c), jnp.float32),
        ],
        scratch_shapes=[pltpu.VMEM((k, c), _CDT)],
    )(m, x, w, add, batch.astype(jnp.int32))


def _gn_apply_body(g, eps, x_ref, b_ref, cnt_ref, sum_ref, sq_ref,
                   gamma_ref, beta_ref, alpha_ref, out_ref):
    x = x_ref[...]
    b = b_ref[...]
    bm = x.shape[0]
    cnt = jnp.maximum(cnt_ref[...], 1.0)[:, None]
    m = sum_ref[...] / cnt
    alpha = alpha_ref[...]
    var = sq_ref[...] / cnt + m * m * (alpha * alpha - 2.0 * alpha)[None, :]
    rstd = 1.0 / jnp.sqrt(var + eps)
    st = (b[:, None] == jax.lax.broadcasted_iota(jnp.int32, (bm, g), 1)
          ).astype(jnp.float32)
    xc = x - jnp.dot(st, alpha[None, :] * m, preferred_element_type=jnp.float32)
    scale = jnp.dot(st, rstd * gamma_ref[...][None, :],
                    preferred_element_type=jnp.float32)
    out_ref[...] = xc * scale + beta_ref[...][None, :]


def _gn_apply(x, batch, cnt, ssum, ssq, gamma, beta, alpha, g, eps=1e-5,
              bm=512):
    n, c = x.shape
    return pl.pallas_call(
        functools.partial(_gn_apply_body, g, eps),
        grid=(n // bm,),
        in_specs=[
            pl.BlockSpec((bm, c), lambda i: (i, 0)),
            pl.BlockSpec((bm,), lambda i: (i,)),
            pl.BlockSpec((g,), lambda i: (0,)),
            pl.BlockSpec((g, c), lambda i: (0, 0)),
            pl.BlockSpec((g, c), lambda i: (0, 0)),
            pl.BlockSpec((c,), lambda i: (0,)),
            pl.BlockSpec((c,), lambda i: (0,)),
            pl.BlockSpec((c,), lambda i: (0,)),
        ],
        out_specs=pl.BlockSpec((bm, c), lambda i: (i, 0)),
        out_shape=jax.ShapeDtypeStruct((n, c), jnp.float32),
    )(x, batch.astype(jnp.int32), cnt, ssum, ssq, gamma, beta, alpha)


def kernel(x_0, x_1, x_2, hodge_laplacian_0, hodge_laplacian_1,
           hodge_laplacian_2, incidence_1, incidence_2, batch, batch_1, y,
           W0, W1, W2, Wa1, Wa2,
           gn1_gamma, gn1_beta, gn1_alpha, gn2_gamma, gn2_beta, gn2_alpha):
    g = y.shape[0]
    x0b = _conv(hodge_laplacian_0, x_0, W0)
    x1b = _conv(hodge_laplacian_1, x_1, W1)
    x2b = _conv(hodge_laplacian_2, x_2, W2)
    x_2_out = x2b
    x1in, c1, s1, q1 = _agg_stats(incidence_2, x_2_out, Wa1, x1b, batch_1, g)
    x_1_out = _gn_apply(x1in, batch_1, c1, s1, q1,
                        gn1_gamma, gn1_beta, gn1_alpha, g)
    x0in, c2, s2, q2 = _agg_stats(incidence_1, x_1_out, Wa2, x0b, batch, g)
    x_0_out = _gn_apply(x0in, batch, c2, s2, q2,
                        gn2_gamma, gn2_beta, gn2_alpha, g)
    return (x_0_out, x_1_out, x_2_out)


# speedup vs baseline: 3.3728x; 1.0939x over previous
import functools

import jax
import jax.numpy as jnp
from jax.experimental import pallas as pl
from jax.experimental.pallas import tpu as pltpu

_CDT = jnp.bfloat16


def _gn_math(x, b, cnt, ssum, ssq, gamma, beta, alpha, g, eps=1e-5):
    n = x.shape[0]
    cnt = jnp.maximum(cnt, 1.0)[:, None]
    m = ssum / cnt
    var = ssq / cnt + m * m * (alpha * alpha - 2.0 * alpha)[None, :]
    rstd = 1.0 / jnp.sqrt(var + eps)
    st = (b[:, None] == jax.lax.broadcasted_iota(jnp.int32, (n, g), 1)
          ).astype(jnp.float32)
    xc = x - jnp.dot(st, alpha[None, :] * m, preferred_element_type=jnp.float32)
    scale = jnp.dot(st, rstd * gamma[None, :],
                    preferred_element_type=jnp.float32)
    return xc * scale + beta[None, :]


def _conv_body(bm, r, m_ref, x_ref, w_ref, out_ref,
               s_scr, inv_scr, z_scr, cache_scr):
    p = pl.program_id(0)
    i = pl.program_id(1)

    @pl.when(p == 0)
    def _():
        blk = m_ref[...]
        s_scr[pl.ds(i * bm, bm)] = jnp.sum(jnp.abs(blk), axis=1)

        @pl.when(i < r)
        def _():
            cache_scr[pl.ds(i * bm, bm), :] = blk.astype(_CDT)

    @pl.when(p == 1)
    def _():
        @pl.when(i == 0)
        def _():
            s = s_scr[...]
            safe = jnp.where(s != 0, s, 1.0)
            inv = jnp.where(s != 0, 1.0 / jnp.sqrt(safe), 0.0)
            inv_scr[...] = inv
            z = jnp.dot(x_ref[...].astype(_CDT), w_ref[...].astype(_CDT),
                        preferred_element_type=jnp.float32)
            z_scr[...] = (inv[:, None] * z).astype(_CDT)

        invm = inv_scr[pl.ds(i * bm, bm)][:, None]

        @pl.when(i < r)
        def _():
            acc = jnp.dot(cache_scr[pl.ds(i * bm, bm), :], z_scr[...],
                          preferred_element_type=jnp.float32)
            out_ref[...] = jnp.maximum(invm * acc, 0.0)

        @pl.when(i >= r)
        def _():
            acc = jnp.dot(m_ref[...].astype(_CDT), z_scr[...],
                          preferred_element_type=jnp.float32)
            out_ref[...] = jnp.maximum(invm * acc, 0.0)


def _conv(m, x, w, bm=512, r=4):
    n = m.shape[0]
    c = x.shape[1]
    nblk = n // bm
    r = min(r, nblk)
    last = min(r, nblk - 1)
    return pl.pallas_call(
        functools.partial(_conv_body, bm, r),
        grid=(2, nblk),
        in_specs=[
            pl.BlockSpec((bm, n),
                         lambda p, i: (jnp.where(p == 0, i,
                                                 jnp.maximum(i, last)), 0)),
            pl.BlockSpec((n, c), lambda p, i: (0, 0)),
            pl.BlockSpec((c, c), lambda p, i: (0, 0)),
        ],
        out_specs=pl.BlockSpec((bm, c),
                               lambda p, i: (jnp.where(p == 0, 0, i), 0)),
        out_shape=jax.ShapeDtypeStruct((n, c), jnp.float32),
        scratch_shapes=[
            pltpu.VMEM((n,), jnp.float32),
            pltpu.VMEM((n,), jnp.float32),
            pltpu.VMEM((n, c), _CDT),
            pltpu.VMEM((r * bm, n), _CDT),
        ],
    )(m, x, w)


def _agg_stats_body(g, gn, m_ref, x_ref, w_ref, add_ref, b_ref, *rest):
    if gn:
        (bx_ref, cnt1_ref, sum1_ref, sq1_ref, gam_ref, bet_ref, alp_ref,
         out_ref, xn_ref, cnt_ref, sum_ref, sq_ref, z_scr) = rest
    else:
        out_ref, cnt_ref, sum_ref, sq_ref, z_scr = rest
    i = pl.program_id(0)

    @pl.when(i == 0)
    def _():
        if gn:
            xn = _gn_math(x_ref[...], bx_ref[...], cnt1_ref[...],
                          sum1_ref[...], sq1_ref[...], gam_ref[...],
                          bet_ref[...], alp_ref[...], g)
            xn_ref[...] = xn
        else:
            xn = x_ref[...]
        z_scr[...] = jnp.dot(xn.astype(_CDT), w_ref[...].astype(_CDT),
                             preferred_element_type=jnp.float32).astype(_CDT)
        cnt_ref[...] = jnp.zeros_like(cnt_ref)
        sum_ref[...] = jnp.zeros_like(sum_ref)
        sq_ref[...] = jnp.zeros_like(sq_ref)

    acc = jnp.dot(m_ref[...].astype(_CDT), z_scr[...],
                  preferred_element_type=jnp.float32)
    v = add_ref[...] + acc
    out_ref[...] = v
    b = b_ref[...]
    bm = b.shape[0]
    sg = (jax.lax.broadcasted_iota(jnp.int32, (g, bm), 0)
          == b[None, :]).astype(jnp.float32)
    cnt_ref[...] += jnp.sum(sg, axis=1)
    sum_ref[...] += jnp.dot(sg, v, preferred_element_type=jnp.float32)
    sq_ref[...] += jnp.dot(sg, v * v, preferred_element_type=jnp.float32)


def _agg_stats(m, x, w, add, batch, g, gn_args=None, bm=512):
    n, k = m.shape
    c = x.shape[1]
    gn = gn_args is not None
    in_specs = [
        pl.BlockSpec((bm, k), lambda i: (i, 0)),
        pl.BlockSpec((k, c), lambda i: (0, 0)),
        pl.BlockSpec((c, c), lambda i: (0, 0)),
        pl.BlockSpec((bm, c), lambda i: (i, 0)),
        pl.BlockSpec((bm,), lambda i: (i,)),
    ]
    args = [m, x, w, add, batch.astype(jnp.int32)]
    out_specs = [
        pl.BlockSpec((bm, c), lambda i: (i, 0)),
        pl.BlockSpec((g,), lambda i: (0,)),
        pl.BlockSpec((g, c), lambda i: (0, 0)),
        pl.BlockSpec((g, c), lambda i: (0, 0)),
    ]
    out_shape = [
        jax.ShapeDtypeStruct((n, c), jnp.float32),
        jax.ShapeDtypeStruct((g,), jnp.float32),
        jax.ShapeDtypeStruct((g, c), jnp.float32),
        jax.ShapeDtypeStruct((g, c), jnp.float32),
    ]
    if gn:
        bx, cnt1, sum1, sq1, gam, bet, alp = gn_args
        in_specs += [
            pl.BlockSpec((k,), lambda i: (0,)),
            pl.BlockSpec((g,), lambda i: (0,)),
            pl.BlockSpec((g, c), lambda i: (0, 0)),
            pl.BlockSpec((g, c), lambda i: (0, 0)),
            pl.BlockSpec((c,), lambda i: (0,)),
            pl.BlockSpec((c,), lambda i: (0,)),
            pl.BlockSpec((c,), lambda i: (0,)),
        ]
        args += [bx.astype(jnp.int32), cnt1, sum1, sq1, gam, bet, alp]
        out_specs.insert(1, pl.BlockSpec((k, c), lambda i: (0, 0)))
        out_shape.insert(1, jax.ShapeDtypeStruct((k, c), jnp.float32))
    return pl.pallas_call(
        functools.partial(_agg_stats_body, g, gn),
        grid=(n // bm,),
        in_specs=in_specs,
        out_specs=out_specs,
        out_shape=out_shape,
        scratch_shapes=[pltpu.VMEM((k, c), _CDT)],
    )(*args)


def _gn_apply_body(g, x_ref, b_ref, cnt_ref, sum_ref, sq_ref,
                   gamma_ref, beta_ref, alpha_ref, out_ref):
    out_ref[...] = _gn_math(x_ref[...], b_ref[...], cnt_ref[...],
                            sum_ref[...], sq_ref[...], gamma_ref[...],
                            beta_ref[...], alpha_ref[...], g)


def _gn_apply(x, batch, cnt, ssum, ssq, gamma, beta, alpha, g, bm=512):
    n, c = x.shape
    return pl.pallas_call(
        functools.partial(_gn_apply_body, g),
        grid=(n // bm,),
        in_specs=[
            pl.BlockSpec((bm, c), lambda i: (i, 0)),
            pl.BlockSpec((bm,), lambda i: (i,)),
            pl.BlockSpec((g,), lambda i: (0,)),
            pl.BlockSpec((g, c), lambda i: (0, 0)),
            pl.BlockSpec((g, c), lambda i: (0, 0)),
            pl.BlockSpec((c,), lambda i: (0,)),
            pl.BlockSpec((c,), lambda i: (0,)),
            pl.BlockSpec((c,), lambda i: (0,)),
        ],
        out_specs=pl.BlockSpec((bm, c), lambda i: (i, 0)),
        out_shape=jax.ShapeDtypeStruct((n, c), jnp.float32),
    )(x, batch.astype(jnp.int32), cnt, ssum, ssq, gamma, beta, alpha)


def kernel(x_0, x_1, x_2, hodge_laplacian_0, hodge_laplacian_1,
           hodge_laplacian_2, incidence_1, incidence_2, batch, batch_1, y,
           W0, W1, W2, Wa1, Wa2,
           gn1_gamma, gn1_beta, gn1_alpha, gn2_gamma, gn2_beta, gn2_alpha):
    g = y.shape[0]
    x0b = _conv(hodge_laplacian_0, x_0, W0)
    x1b = _conv(hodge_laplacian_1, x_1, W1)
    x2b = _conv(hodge_laplacian_2, x_2, W2)
    x_2_out = x2b
    x1in, c1, s1, q1 = _agg_stats(incidence_2, x_2_out, Wa1, x1b, batch_1, g)
    x0in, x_1_out, c2, s2, q2 = _agg_stats(
        incidence_1, x1in, Wa2, x0b, batch, g,
        gn_args=(batch_1, c1, s1, q1, gn1_gamma, gn1_beta, gn1_alpha))
    x_0_out = _gn_apply(x0in, batch, c2, s2, q2,
                        gn2_gamma, gn2_beta, gn2_alpha, g)
    return (x_0_out, x_1_out, x_2_out)


# speedup vs baseline: 3.4939x; 1.0359x over previous
import functools

import jax
import jax.numpy as jnp
from jax.experimental import pallas as pl
from jax.experimental.pallas import tpu as pltpu

_CDT = jnp.bfloat16


def _conv_body(bm, r, m_ref, x_ref, w_ref, out_ref,
               s_scr, inv_scr, z_scr, cache_scr):
    p = pl.program_id(0)
    i = pl.program_id(1)

    @pl.when(p == 0)
    def _():
        blk = m_ref[...]
        s_scr[pl.ds(i * bm, bm)] = jnp.sum(jnp.abs(blk), axis=1)

        @pl.when(i < r)
        def _():
            cache_scr[pl.ds(i * bm, bm), :] = blk.astype(_CDT)

    @pl.when(p == 1)
    def _():
        @pl.when(i == 0)
        def _():
            s = s_scr[...]
            safe = jnp.where(s != 0, s, 1.0)
            inv = jnp.where(s != 0, 1.0 / jnp.sqrt(safe), 0.0)
            inv_scr[...] = inv
            z = jnp.dot(x_ref[...].astype(_CDT), w_ref[...].astype(_CDT),
                        preferred_element_type=jnp.float32)
            z_scr[...] = (inv[:, None] * z).astype(_CDT)

        invm = inv_scr[pl.ds(i * bm, bm)][:, None]

        @pl.when(i < r)
        def _():
            acc = jnp.dot(cache_scr[pl.ds(i * bm, bm), :], z_scr[...],
                          preferred_element_type=jnp.float32)
            out_ref[...] = jnp.maximum(invm * acc, 0.0)

        @pl.when(i >= r)
        def _():
            acc = jnp.dot(m_ref[...].astype(_CDT), z_scr[...],
                          preferred_element_type=jnp.float32)
            out_ref[...] = jnp.maximum(invm * acc, 0.0)


def _conv(m, x, w, bm=512, r=6):
    n = m.shape[0]
    c = x.shape[1]
    nblk = n // bm
    r = min(r, nblk)
    last = min(r, nblk - 1)
    return pl.pallas_call(
        functools.partial(_conv_body, bm, r),
        grid=(2, nblk),
        in_specs=[
            pl.BlockSpec((bm, n),
                         lambda p, i: (jnp.where(p == 0, i,
                                                 jnp.maximum(i, last)), 0)),
            pl.BlockSpec((n, c), lambda p, i: (0, 0)),
            pl.BlockSpec((c, c), lambda p, i: (0, 0)),
        ],
        out_specs=pl.BlockSpec((bm, c),
                               lambda p, i: (jnp.where(p == 0, 0, i), 0)),
        out_shape=jax.ShapeDtypeStruct((n, c), jnp.float32),
        scratch_shapes=[
            pltpu.VMEM((n,), jnp.float32),
            pltpu.VMEM((n,), jnp.float32),
            pltpu.VMEM((n, c), _CDT),
            pltpu.VMEM((r * bm, n), _CDT),
        ],
    )(m, x, w)


def _agg_gn_body(g, bm, eps, m_ref, x_ref, w_ref, add_ref, b_ref,
                 gam_ref, bet_ref, alp_ref, out_ref,
                 z_scr, v_scr, cnt_scr, sum_scr, sq_scr):
    p = pl.program_id(0)
    i = pl.program_id(1)

    @pl.when(p == 0)
    def _():
        @pl.when(i == 0)
        def _():
            z_scr[...] = jnp.dot(x_ref[...].astype(_CDT),
                                 w_ref[...].astype(_CDT),
                                 preferred_element_type=jnp.float32
                                 ).astype(_CDT)
            cnt_scr[...] = jnp.zeros_like(cnt_scr)
            sum_scr[...] = jnp.zeros_like(sum_scr)
            sq_scr[...] = jnp.zeros_like(sq_scr)

        acc = jnp.dot(m_ref[...].astype(_CDT), z_scr[...],
                      preferred_element_type=jnp.float32)
        v = add_ref[...] + acc
        v_scr[pl.ds(i * bm, bm), :] = v
        b = b_ref[...]
        sg = (jax.lax.broadcasted_iota(jnp.int32, (g, bm), 0)
              == b[None, :]).astype(jnp.float32)
        cnt_scr[...] += jnp.sum(sg, axis=1)
        sum_scr[...] += jnp.dot(sg, v, preferred_element_type=jnp.float32)
        sq_scr[...] += jnp.dot(sg, v * v, preferred_element_type=jnp.float32)

    @pl.when(p == 1)
    def _():
        v = v_scr[pl.ds(i * bm, bm), :]
        b = b_ref[...]
        alpha = alp_ref[...]
        cnt = jnp.maximum(cnt_scr[...], 1.0)[:, None]
        mean = sum_scr[...] / cnt
        var = (sq_scr[...] / cnt
               + mean * mean * (alpha * alpha - 2.0 * alpha)[None, :])
        rstd = 1.0 / jnp.sqrt(var + eps)
        st = (b[:, None] == jax.lax.broadcasted_iota(jnp.int32, (bm, g), 1)
              ).astype(jnp.float32)
        xc = v - jnp.dot(st, alpha[None, :] * mean,
                         preferred_element_type=jnp.float32)
        scale = jnp.dot(st, rstd * gam_ref[...][None, :],
                        preferred_element_type=jnp.float32)
        out_ref[...] = xc * scale + bet_ref[...][None, :]


def _agg_gn(m, x, w, add, batch, gamma, beta, alpha, g, eps=1e-5, bm=512):
    n, k = m.shape
    c = x.shape[1]
    nblk = n // bm
    return pl.pallas_call(
        functools.partial(_agg_gn_body, g, bm, eps),
        grid=(2, nblk),
        in_specs=[
            pl.BlockSpec((bm, k),
                         lambda p, i: (jnp.where(p == 0, i, nblk - 1), 0)),
            pl.BlockSpec((k, c), lambda p, i: (0, 0)),
            pl.BlockSpec((c, c), lambda p, i: (0, 0)),
            pl.BlockSpec((bm, c),
                         lambda p, i: (jnp.where(p == 0, i, nblk - 1), 0)),
            pl.BlockSpec((bm,), lambda p, i: (i,)),
            pl.BlockSpec((c,), lambda p, i: (0,)),
            pl.BlockSpec((c,), lambda p, i: (0,)),
            pl.BlockSpec((c,), lambda p, i: (0,)),
        ],
        out_specs=pl.BlockSpec((bm, c),
                               lambda p, i: (jnp.where(p == 0, 0, i), 0)),
        out_shape=jax.ShapeDtypeStruct((n, c), jnp.float32),
        scratch_shapes=[
            pltpu.VMEM((k, c), _CDT),
            pltpu.VMEM((n, c), jnp.float32),
            pltpu.VMEM((g,), jnp.float32),
            pltpu.VMEM((g, c), jnp.float32),
            pltpu.VMEM((g, c), jnp.float32),
        ],
    )(m, x, w, add, batch.astype(jnp.int32), gamma, beta, alpha)


def kernel(x_0, x_1, x_2, hodge_laplacian_0, hodge_laplacian_1,
           hodge_laplacian_2, incidence_1, incidence_2, batch, batch_1, y,
           W0, W1, W2, Wa1, Wa2,
           gn1_gamma, gn1_beta, gn1_alpha, gn2_gamma, gn2_beta, gn2_alpha):
    g = y.shape[0]
    x0b = _conv(hodge_laplacian_0, x_0, W0)
    x1b = _conv(hodge_laplacian_1, x_1, W1)
    x2b = _conv(hodge_laplacian_2, x_2, W2)
    x_2_out = x2b
    x_1_out = _agg_gn(incidence_2, x_2_out, Wa1, x1b, batch_1,
                      gn1_gamma, gn1_beta, gn1_alpha, g)
    x_0_out = _agg_gn(incidence_1, x_1_out, Wa2, x0b, batch,
                      gn2_gamma, gn2_beta, gn2_alpha, g)
    return (x_0_out, x_1_out, x_2_out)
